# baseline (speedup 1.0000x reference)
import functools

import jax
import jax.numpy as jnp
from jax import lax
from jax.experimental import pallas as pl
from jax.experimental.pallas import tpu as pltpu
from jax.experimental.pallas import tpu_sc as plsc

_B = 4096
_L = 200
_H = 32
_NC = 2
_NS = 16
_NW = _NC * _NS


def _tok_pos_body(xTr_hbm, emb_hbm, pos_hbm, out_hbm,
                  idx_all, rowsA, rowsB, blkA, blkB, pos_v,
                  sgA, sgB, soA, soB):
    wid = lax.axis_index("s") * _NC + lax.axis_index("c")
    pltpu.sync_copy(pos_hbm, pos_v)
    pltpu.sync_copy(xTr_hbm.at[wid], idx_all)

    iota = lax.iota(jnp.int32, 16)

    def gather_start(q, rowsbuf, sem):
        pltpu.async_copy(
            emb_hbm.at[idx_all.at[pl.ds(q * 512, 512)]], rowsbuf, sem)

    def gather_wait(rowsbuf, sem):
        pltpu.make_async_copy(
            emb_hbm.at[idx_all.at[pl.ds(0, 512)]], rowsbuf, sem).wait()

    def out_start(l, blk, sem):
        pltpu.async_copy(
            blk, out_hbm.at[pl.ds(l * 4, 4), pl.ds(wid, 1), :], sem)

    def out_wait(blk, sem):
        pltpu.make_async_copy(
            blk, out_hbm.at[pl.ds(0, 4), pl.ds(0, 1), :], sem).wait()

    def compute(l, rowsbuf, off, blk):
        p_lo = pos_v[l, pl.ds(0, 16)]
        p_hi = pos_v[l, pl.ds(16, 16)]

        @plsc.parallel_loop(0, 128, unroll=8)
        def _(c):
            rowsbuf[off + c, pl.ds(0, 16)] = (
                rowsbuf[off + c, pl.ds(0, 16)] + p_lo)
            rowsbuf[off + c, pl.ds(16, 16)] = (
                rowsbuf[off + c, pl.ds(16, 16)] + p_hi)

        @plsc.parallel_loop(0, _H, unroll=4)
        def _(h):
            hs = jnp.full((16,), h, jnp.int32)
            g = lax.shift_right_logical(h, 3)
            base = lax.bitwise_and(h, 7) * 128
            for k in range(8):
                blk[g, 0, pl.ds(base + 16 * k, 16)] = plsc.load_gather(
                    rowsbuf, [iota + (off + 16 * k), hs])

    gather_start(0, rowsA, sgA)
    gather_start(1, rowsB, sgB)

    def half(q2, q, rowsbuf, sem, first):
        gather_wait(rowsbuf, sem)
        for j in range(4):
            l = 4 * q + j
            blk, so = (blkA, soA) if j % 2 == 0 else (blkB, soB)
            if first and j < 2:
                @pl.when(q2 > 0)
                def _():
                    out_wait(blk, so)
            else:
                out_wait(blk, so)
            compute(l, rowsbuf, j * 128, blk)
            out_start(l, blk, so)

        @pl.when(q + 2 < _L // 4)
        def _():
            gather_start(q + 2, rowsbuf, sem)

    def body(q2, carry):
        q = 2 * q2
        half(q2, q, rowsA, sgA, True)
        half(q2, q + 1, rowsB, sgB, False)
        return carry

    lax.fori_loop(0, _L // 8, body, 0)
    out_wait(blkA, soA)
    out_wait(blkB, soB)


def _pack_block(in_ref, o_ref):
    xs = in_ref[...]
    for k in range(8):
        q = jnp.concatenate(
            [xs[:, k * 512 + a * 128:k * 512 + (a + 1) * 128]
             for a in range(4)], axis=0)
        o_ref[k * 128:(k + 1) * 128, :] = jnp.swapaxes(q, 0, 1)


def _pack_table(embT):
    grid = (1000000 + 4095) // 4096
    return pl.pallas_call(
        _pack_block,
        grid=(grid,),
        in_specs=[pl.BlockSpec((_H, 4096), lambda i: (0, i))],
        out_specs=pl.BlockSpec((1024, 128), lambda i: (i, 0)),
        out_shape=jax.ShapeDtypeStruct((250880, 128), jnp.float32),
    )(embT)


def kernel(x, emb_table, pos_table):
    xTr = x.T.reshape(_L, _NW, 128).transpose(1, 0, 2).reshape(_NW, _L * 128)
    xTr = ((xTr & ~jnp.int32(511)) | ((xTr & 127) << 2) | ((xTr >> 7) & 3))
    emb_rows = _pack_table(emb_table.T).reshape(1003520, _H)
    mesh = plsc.VectorSubcoreMesh(core_axis_name="c", subcore_axis_name="s")
    call = functools.partial(
        pl.kernel,
        mesh=mesh,
        compiler_params=pltpu.CompilerParams(
            use_tc_tiling_on_sc=False, needs_layout_passes=False),
        out_type=jax.ShapeDtypeStruct((_L * 4, _NW, 1024), jnp.float32),
        scratch_types=[
            pltpu.VMEM((_L * 128,), jnp.int32),
            pltpu.VMEM((512, _H), jnp.float32),
            pltpu.VMEM((512, _H), jnp.float32),
            pltpu.VMEM((4, 1, 1024), jnp.float32),
            pltpu.VMEM((4, 1, 1024), jnp.float32),
            pltpu.VMEM((_L, _H), jnp.float32),
            pltpu.SemaphoreType.DMA,
            pltpu.SemaphoreType.DMA,
            pltpu.SemaphoreType.DMA,
            pltpu.SemaphoreType.DMA,
        ],
    )(_tok_pos_body)
    out = call(xTr, emb_rows, pos_table)
    v = out.reshape(_L, 4, _NW, 8, 128)
    return v.transpose(2, 4, 0, 1, 3).reshape(_B, _L, _H)


# speedup vs baseline: 1.0055x; 1.0055x over previous
import functools

import jax
import jax.numpy as jnp
from jax import lax
from jax.experimental import pallas as pl
from jax.experimental.pallas import tpu as pltpu
from jax.experimental.pallas import tpu_sc as plsc

_B = 4096
_L = 200
_H = 32
_NC = 2
_NS = 16
_NW = _NC * _NS


def _tok_pos_body(xTr_hbm, emb_hbm, pos_hbm, out_hbm,
                  idx_all, rows0, rows1, rows2, rows3, blkA, blkB, pos_v,
                  sg0, sg1, sg2, sg3, soA, soB):
    wid = lax.axis_index("s") * _NC + lax.axis_index("c")
    pltpu.sync_copy(pos_hbm, pos_v)
    pltpu.sync_copy(xTr_hbm.at[wid], idx_all)

    iota = lax.iota(jnp.int32, 16)
    rows = (rows0, rows1, rows2, rows3)
    sg = (sg0, sg1, sg2, sg3)
    _NQ = _L // 4

    def gather_start(q, rowsbuf, sem):
        pltpu.async_copy(
            emb_hbm.at[idx_all.at[pl.ds(q * 512, 512)]], rowsbuf, sem)

    def gather_wait(rowsbuf, sem):
        pltpu.make_async_copy(
            emb_hbm.at[idx_all.at[pl.ds(0, 512)]], rowsbuf, sem).wait()

    def out_start(l, blk, sem):
        pltpu.async_copy(
            blk, out_hbm.at[pl.ds(l * 4, 4), pl.ds(wid, 1), :], sem)

    def out_wait(blk, sem):
        pltpu.make_async_copy(
            blk, out_hbm.at[pl.ds(0, 4), pl.ds(0, 1), :], sem).wait()

    def compute(l, rowsbuf, off, blk):
        ls = jnp.full((16,), l, jnp.int32)

        @plsc.parallel_loop(0, _H, unroll=4)
        def _(h):
            hs = jnp.full((16,), h, jnp.int32)
            ps = plsc.load_gather(pos_v, [ls, hs])
            g = lax.shift_right_logical(h, 3)
            base = lax.bitwise_and(h, 7) * 128
            for k in range(8):
                blk[g, 0, pl.ds(base + 16 * k, 16)] = plsc.load_gather(
                    rowsbuf, [iota + (off + 16 * k), hs]) + ps

    def do_stream(q, rowsbuf, sem, guard):
        gather_wait(rowsbuf, sem)
        for j2 in range(4):
            l = 4 * q + j2
            blk, so = (blkA, soA) if j2 % 2 == 0 else (blkB, soB)
            if guard is not None and j2 < 2:
                @pl.when(guard)
                def _():
                    out_wait(blk, so)
            else:
                out_wait(blk, so)
            compute(l, rowsbuf, j2 * 128, blk)
            out_start(l, blk, so)

    for j in range(4):
        gather_start(j, rows[j], sg[j])

    def body(i, carry):
        for j in range(4):
            q = 4 * i + j
            do_stream(q, rows[j], sg[j], (i > 0) if j == 0 else None)

            @pl.when(q + 4 < _NQ)
            def _():
                gather_start(q + 4, rows[j], sg[j])
        return carry

    lax.fori_loop(0, _NQ // 4, body, 0)
    do_stream(_NQ - 2, rows[0], sg[0], None)
    do_stream(_NQ - 1, rows[1], sg[1], None)
    out_wait(blkA, soA)
    out_wait(blkB, soB)


def _pack_block(in_ref, o_ref):
    xs = in_ref[...]
    for k in range(8):
        q = jnp.concatenate(
            [xs[:, k * 512 + a * 128:k * 512 + (a + 1) * 128]
             for a in range(4)], axis=0)
        o_ref[k * 128:(k + 1) * 128, :] = jnp.swapaxes(q, 0, 1)


def _pack_table(embT):
    grid = (1000000 + 4095) // 4096
    return pl.pallas_call(
        _pack_block,
        grid=(grid,),
        in_specs=[pl.BlockSpec((_H, 4096), lambda i: (0, i))],
        out_specs=pl.BlockSpec((1024, 128), lambda i: (i, 0)),
        out_shape=jax.ShapeDtypeStruct((250880, 128), jnp.float32),
    )(embT)


def kernel(x, emb_table, pos_table):
    xTr = x.T.reshape(_L, _NW, 128).transpose(1, 0, 2).reshape(_NW, _L * 128)
    xTr = ((xTr & ~jnp.int32(511)) | ((xTr & 127) << 2) | ((xTr >> 7) & 3))
    emb_rows = _pack_table(emb_table.T).reshape(1003520, _H)
    mesh = plsc.VectorSubcoreMesh(core_axis_name="c", subcore_axis_name="s")
    call = functools.partial(
        pl.kernel,
        mesh=mesh,
        compiler_params=pltpu.CompilerParams(
            use_tc_tiling_on_sc=False, needs_layout_passes=False),
        out_type=jax.ShapeDtypeStruct((_L * 4, _NW, 1024), jnp.float32),
        scratch_types=[
            pltpu.VMEM((_L * 128,), jnp.int32),
            pltpu.VMEM((512, _H), jnp.float32),
            pltpu.VMEM((512, _H), jnp.float32),
            pltpu.VMEM((512, _H), jnp.float32),
            pltpu.VMEM((512, _H), jnp.float32),
            pltpu.VMEM((4, 1, 1024), jnp.float32),
            pltpu.VMEM((4, 1, 1024), jnp.float32),
            pltpu.VMEM((_L, _H), jnp.float32),
            pltpu.SemaphoreType.DMA,
            pltpu.SemaphoreType.DMA,
            pltpu.SemaphoreType.DMA,
            pltpu.SemaphoreType.DMA,
            pltpu.SemaphoreType.DMA,
            pltpu.SemaphoreType.DMA,
        ],
    )(_tok_pos_body)
    out = call(xTr, emb_rows, pos_table)
    v = out.reshape(_L, 4, _NW, 8, 128)
    return v.transpose(2, 4, 0, 1, 3).reshape(_B, _L, _H)
